# Initial kernel scaffold; baseline (speedup 1.0000x reference)
import functools
import jax
import jax.numpy as jnp
import numpy as np
from jax.experimental import pallas as pl
from jax.experimental.pallas import tpu as pltpu

KS = 4
NCELL = KS * KS * KS
EXTENT = float(np.float32(1.5 * 6 * 0.025))
RADIUS = 0.5 * EXTENT
TILE = 128
MAXC = 16


def _conv_body(cnt_ref, tbl_ref, posr_ref, posc_ref, feat_ref, w_ref,
               out_ref, acc_ref, *, n_real, out_ch):
    r = pl.program_id(0)
    c = pl.program_id(1)

    @pl.when(c == 0)
    def _():
        acc_ref[...] = jnp.zeros_like(acc_ref)

    @pl.when(c < cnt_ref[r])
    def _():
        inv_r = 1.0 / RADIUS
        prx = posr_ref[:, 0:1]
        pry = posr_ref[:, 1:2]
        prz = posr_ref[:, 2:3]
        pcx = posc_ref[0:1, :]
        pcy = posc_ref[1:2, :]
        pcz = posc_ref[2:3, :]
        rx = (pcx - prx) * inv_r
        ry = (pcy - pry) * inv_r
        rz = (pcz - prz) * inv_r
        r2 = rx * rx + ry * ry + rz * rz
        gi = r * TILE + jax.lax.broadcasted_iota(jnp.int32, (TILE, TILE), 0)
        gj = tbl_ref[r, c] * TILE + jax.lax.broadcasted_iota(
            jnp.int32, (TILE, TILE), 1)
        mask = (r2 <= 1.0) & (gi != gj) & (gj < n_real)
        w1 = 1.0 - r2
        wm = jnp.where(mask, w1 * w1 * w1, 0.0)
        norm2 = jnp.sqrt(jnp.maximum(r2, 1e-12))
        norminf = jnp.maximum(jnp.maximum(jnp.abs(rx), jnp.abs(ry)),
                              jnp.abs(rz))
        scale = jnp.where(norminf > 1e-12,
                          norm2 / jnp.maximum(norminf, 1e-12), 0.0)

        def axis_weights(rel):
            t = (rel * scale + 1.0) * (0.5 * (KS - 1))
            c0 = jnp.minimum(jnp.maximum(jnp.floor(t), 0.0), float(KS - 2))
            f = t - c0
            ws = []
            for k in range(KS):
                w = jnp.where(c0 == float(k), 1.0 - f, 0.0)
                if k >= 1:
                    w = w + jnp.where(c0 == float(k - 1), f, 0.0)
                ws.append(w)
            return ws

        wxs = axis_weights(rx)
        wys = axis_weights(ry)
        wzs = axis_weights(rz)
        wxs = [w * wm for w in wxs]
        feat = feat_ref[...]
        for kx in range(KS):
            for ky in range(KS):
                m = wxs[kx] * wys[ky]
                for kz in range(KS):
                    cell = (kx * KS + ky) * KS + kz
                    wk = m * wzs[kz]
                    acc_ref[cell] = acc_ref[cell] + jax.lax.dot(
                        wk, feat, preferred_element_type=jnp.float32)

    @pl.when(c == MAXC - 1)
    def _():
        o = jnp.zeros((TILE, out_ch), jnp.float32)
        for cell in range(NCELL):
            o = o + jax.lax.dot(acc_ref[cell], w_ref[cell],
                                preferred_element_type=jnp.float32)
        out_ref[...] = o


def _cconv(cnt, tbl, pos_rows, pos_colsT, feat_pad, wflat, n_real):
    npad, ch = feat_pad.shape
    nrow = npad // TILE
    out_ch = wflat.shape[2]
    body = functools.partial(_conv_body, n_real=n_real, out_ch=out_ch)
    grid_spec = pltpu.PrefetchScalarGridSpec(
        num_scalar_prefetch=2,
        grid=(nrow, MAXC),
        in_specs=[
            pl.BlockSpec((TILE, 3), lambda r, c, cnt, tbl: (r, 0)),
            pl.BlockSpec((3, TILE), lambda r, c, cnt, tbl: (0, tbl[r, c])),
            pl.BlockSpec((TILE, ch), lambda r, c, cnt, tbl: (tbl[r, c], 0)),
            pl.BlockSpec((NCELL, ch, out_ch),
                         lambda r, c, cnt, tbl: (0, 0, 0)),
        ],
        out_specs=pl.BlockSpec((TILE, out_ch), lambda r, c, cnt, tbl: (r, 0)),
        scratch_shapes=[pltpu.VMEM((NCELL, TILE, ch), jnp.float32)],
    )
    return pl.pallas_call(
        body,
        grid_spec=grid_spec,
        out_shape=jax.ShapeDtypeStruct((npad, out_ch), jnp.float32),
        compiler_params=pltpu.CompilerParams(
            dimension_semantics=("arbitrary", "arbitrary")),
    )(cnt, tbl, pos_rows, pos_colsT, feat_pad, wflat)


def _bn(xv, g, be, eps=1e-5):
    m = jnp.mean(xv, axis=0)
    v = jnp.mean((xv - m) ** 2, axis=0)
    return (xv - m) / jnp.sqrt(v + eps) * g + be


def _morton_order(pos):
    pmin = pos.min(axis=0)
    span = jnp.maximum(pos.max(axis=0) - pmin, 1e-9)
    cell = jnp.clip(((pos - pmin) / span * 16.0).astype(jnp.int32), 0, 15)
    code = jnp.zeros(pos.shape[0], jnp.int32)
    for b in range(4):
        code = (code
                | (((cell[:, 0] >> b) & 1) << (3 * b + 2))
                | (((cell[:, 1] >> b) & 1) << (3 * b + 1))
                | (((cell[:, 2] >> b) & 1) << (3 * b)))
    return jnp.argsort(code)


def _pair_table(pos_pad, n_real):
    npad = pos_pad.shape[0]
    nrow = npad // TILE
    valid = (jnp.arange(npad) < n_real)[:, None]
    big = jnp.float32(3e4)
    plo = jnp.where(valid, pos_pad, big).reshape(nrow, TILE, 3).min(axis=1)
    phi = jnp.where(valid, pos_pad, -big).reshape(nrow, TILE, 3).max(axis=1)
    gap = jnp.maximum(
        jnp.maximum(plo[None, :, :] - phi[:, None, :],
                    plo[:, None, :] - phi[None, :, :]), 0.0)
    d2 = jnp.sum(gap * gap, axis=-1)
    active = d2 <= (RADIUS * RADIUS)
    cnt = active.sum(axis=1).astype(jnp.int32)
    tbl = jnp.argsort(~active, axis=1, stable=True)[:, :MAXC].astype(jnp.int32)
    return cnt, tbl


def kernel(x, y, pos, W1, b1, g1, be1, W2, b2, g2, be2):
    n = x.shape[0]
    ch1 = 2 * x.shape[1]
    inter = W1.shape[-1]
    out_ch = W2.shape[-1]
    npad = ((n + TILE - 1) // TILE) * TILE

    order = _morton_order(pos)
    pos_s = pos[order]
    xs = x[order]
    ys = y[order]

    pad = npad - n
    pos_pad = jnp.concatenate(
        [pos_s, jnp.full((pad, 3), 1e4, jnp.float32)], axis=0)
    cnt, tbl = _pair_table(pos_pad, n)
    pos_colsT = pos_pad.T

    w1flat = W1.reshape(NCELL, ch1, inter)
    w2flat = W2.reshape(NCELL, inter, out_ch)

    feat1 = jnp.concatenate([xs, ys], axis=1)
    feat1 = jnp.concatenate([feat1, jnp.zeros((pad, ch1), feat1.dtype)], 0)
    xl = _cconv(cnt, tbl, pos_pad, pos_colsT, feat1, w1flat, n)[:n] + b1
    xl = jax.nn.relu(_bn(xl, g1, be1))

    feat2 = jnp.concatenate([xl, jnp.zeros((pad, inter), xl.dtype)], 0)
    xl2 = _cconv(cnt, tbl, pos_pad, pos_colsT, feat2, w2flat, n)[:n] + b2
    wei = jax.nn.sigmoid(_bn(xl2, g2, be2))

    res = 2.0 * xs * wei + 2.0 * ys * (1.0 - wei)
    return jnp.zeros((n, x.shape[1]), res.dtype).at[order].set(res)


# speedup vs baseline: 24.1633x; 24.1633x over previous
import functools
import jax
import jax.numpy as jnp
import numpy as np
from jax.experimental import pallas as pl
from jax.experimental.pallas import tpu as pltpu

KS = 4
NCELL = KS * KS * KS
EXTENT = float(np.float32(1.5 * 6 * 0.025))
RADIUS = 0.5 * EXTENT
TILE = 128
MAXC = 28


def _conv_body(cnt_ref, tbl_ref, posr_ref, posc_ref, feat_ref, w_ref,
               out_ref, acc_ref, *, n_real, out_ch, maxc):
    r = pl.program_id(0)
    c = pl.program_id(1)

    @pl.when(c == 0)
    def _():
        acc_ref[...] = jnp.zeros_like(acc_ref)

    @pl.when(c < cnt_ref[r])
    def _():
        radius = jnp.float32(RADIUS)
        prx = posr_ref[:, 0:1]
        pry = posr_ref[:, 1:2]
        prz = posr_ref[:, 2:3]
        pcx = posc_ref[0:1, :]
        pcy = posc_ref[1:2, :]
        pcz = posc_ref[2:3, :]
        rx = (pcx - prx) / radius
        ry = (pcy - pry) / radius
        rz = (pcz - prz) / radius
        r2 = rx * rx + ry * ry + rz * rz
        gi = r * TILE + jax.lax.broadcasted_iota(jnp.int32, (TILE, TILE), 0)
        gj = tbl_ref[r, c] * TILE + jax.lax.broadcasted_iota(
            jnp.int32, (TILE, TILE), 1)
        sqr = prx * prx + pry * pry + prz * prz
        sqc = pcx * pcx + pcy * pcy + pcz * pcz
        pr3 = posr_ref[:, 0:3].astype(jnp.bfloat16)
        pc3 = posc_ref[0:3, :].astype(jnp.bfloat16)
        dotm = jax.lax.dot(pr3, pc3, preferred_element_type=jnp.float32)
        d2 = sqr + sqc - 2.0 * dotm
        mask = (d2 <= radius * radius) & (gi != gj) & (gj < n_real)

        @pl.when(jnp.any(mask))
        def _():
            w1 = 1.0 - r2
            w3 = jnp.clip(w1 * w1 * w1, 0.0, 1.0)
            wm = jnp.where(mask, w3, 0.0)
            norm2 = jnp.sqrt(jnp.maximum(r2, 1e-12))
            norminf = jnp.maximum(jnp.maximum(jnp.abs(rx), jnp.abs(ry)),
                                  jnp.abs(rz))
            scale = jnp.where(norminf > 1e-12,
                              norm2 / jnp.maximum(norminf, 1e-12), 0.0)

            def axis_weights(rel):
                t = ((rel * scale + 1.0) * 0.5) * float(KS - 1)
                c0 = jnp.minimum(jnp.maximum(jnp.floor(t), 0.0),
                                 float(KS - 2))
                f = t - c0
                ws = []
                for k in range(KS):
                    w = jnp.where(c0 == float(k), 1.0 - f, 0.0)
                    if k >= 1:
                        w = w + jnp.where(c0 == float(k - 1), f, 0.0)
                    ws.append(w)
                return ws

            wxs = [(w * wm).astype(jnp.bfloat16) for w in axis_weights(rx)]
            wys = [w.astype(jnp.bfloat16) for w in axis_weights(ry)]
            wzs = [w.astype(jnp.bfloat16) for w in axis_weights(rz)]
            feat = feat_ref[...].astype(jnp.bfloat16)
            for kx in range(KS):
                for ky in range(KS):
                    m = wxs[kx] * wys[ky]
                    for kz in range(KS):
                        cell = (kx * KS + ky) * KS + kz
                        wk = m * wzs[kz]
                        acc_ref[cell] = acc_ref[cell] + jax.lax.dot(
                            wk, feat, preferred_element_type=jnp.float32)

    @pl.when(c == maxc - 1)
    def _():
        o = jnp.zeros((TILE, out_ch), jnp.float32)
        for cell in range(NCELL):
            o = o + jax.lax.dot(acc_ref[cell].astype(jnp.bfloat16),
                                w_ref[cell].astype(jnp.bfloat16),
                                preferred_element_type=jnp.float32)
        out_ref[...] = o


def _cconv(cnt, tbl, pos_rows, pos_colsT, feat_pad, wflat, n_real):
    npad, ch = feat_pad.shape
    nrow = npad // TILE
    maxc = tbl.shape[1]
    out_ch = wflat.shape[2]
    body = functools.partial(_conv_body, n_real=n_real, out_ch=out_ch,
                             maxc=maxc)
    grid_spec = pltpu.PrefetchScalarGridSpec(
        num_scalar_prefetch=2,
        grid=(nrow, maxc),
        in_specs=[
            pl.BlockSpec((TILE, 3), lambda r, c, cnt, tbl: (r, 0)),
            pl.BlockSpec((3, TILE), lambda r, c, cnt, tbl: (0, tbl[r, c])),
            pl.BlockSpec((TILE, ch), lambda r, c, cnt, tbl: (tbl[r, c], 0)),
            pl.BlockSpec((NCELL, ch, out_ch),
                         lambda r, c, cnt, tbl: (0, 0, 0)),
        ],
        out_specs=pl.BlockSpec((TILE, out_ch), lambda r, c, cnt, tbl: (r, 0)),
        scratch_shapes=[pltpu.VMEM((NCELL, TILE, ch), jnp.float32)],
    )
    return pl.pallas_call(
        body,
        grid_spec=grid_spec,
        out_shape=jax.ShapeDtypeStruct((npad, out_ch), jnp.float32),
        compiler_params=pltpu.CompilerParams(
            dimension_semantics=("arbitrary", "arbitrary")),
    )(cnt, tbl, pos_rows, pos_colsT, feat_pad, wflat)


def _bn(xv, g, be, eps=1e-5):
    m = jnp.mean(xv, axis=0)
    v = jnp.mean((xv - m) ** 2, axis=0)
    return (xv - m) / jnp.sqrt(v + eps) * g + be


def _snake_order(pos):
    res = 32
    pmin = pos.min(axis=0)
    span = jnp.maximum(pos.max(axis=0) - pmin, 1e-9)
    cell = jnp.clip(((pos - pmin) / span * res).astype(jnp.int32), 0, res - 1)
    cx, cy, cz = cell[:, 0], cell[:, 1], cell[:, 2]
    cy = jnp.where((cx & 1) == 1, res - 1 - cy, cy)
    cz = jnp.where(((cx + cy) & 1) == 1, res - 1 - cz, cz)
    code = (cx << 10) | (cy << 5) | cz
    return jnp.argsort(code)


def _pair_table(pos_pad, n_real):
    npad = pos_pad.shape[0]
    nrow = npad // TILE
    maxc = min(MAXC, nrow)
    valid = (jnp.arange(npad) < n_real)[:, None]
    big = jnp.float32(3e4)
    plo = jnp.where(valid, pos_pad, big).reshape(nrow, TILE, 3).min(axis=1)
    phi = jnp.where(valid, pos_pad, -big).reshape(nrow, TILE, 3).max(axis=1)
    gap = jnp.maximum(
        jnp.maximum(plo[None, :, :] - phi[:, None, :],
                    plo[:, None, :] - phi[None, :, :]), 0.0)
    d2 = jnp.sum(gap * gap, axis=-1)
    active = d2 <= (RADIUS * RADIUS)
    cnt = active.sum(axis=1).astype(jnp.int32)
    tbl = jnp.argsort(~active, axis=1, stable=True)[:, :maxc].astype(jnp.int32)
    tbl = jnp.where(jnp.arange(maxc)[None, :] < cnt[:, None], tbl, tbl[:, :1])
    return cnt, tbl


def kernel(x, y, pos, W1, b1, g1, be1, W2, b2, g2, be2):
    n = x.shape[0]
    ch1 = 2 * x.shape[1]
    inter = W1.shape[-1]
    out_ch = W2.shape[-1]
    npad = ((n + TILE - 1) // TILE) * TILE

    order = _snake_order(pos)
    pos_s = pos[order]
    xs = x[order]
    ys = y[order]

    pad = npad - n
    pos_pad = jnp.concatenate(
        [pos_s, jnp.full((pad, 3), 1e4, jnp.float32)], axis=0)
    cnt, tbl = _pair_table(pos_pad, n)
    pos_colsT = pos_pad.T

    w1flat = W1.reshape(NCELL, ch1, inter)
    w2flat = W2.reshape(NCELL, inter, out_ch)

    feat1 = jnp.concatenate([xs, ys], axis=1)
    feat1 = jnp.concatenate([feat1, jnp.zeros((pad, ch1), feat1.dtype)], 0)
    xl = _cconv(cnt, tbl, pos_pad, pos_colsT, feat1, w1flat, n)[:n] + b1
    xl = jax.nn.relu(_bn(xl, g1, be1))

    feat2 = jnp.concatenate([xl, jnp.zeros((pad, inter), xl.dtype)], 0)
    xl2 = _cconv(cnt, tbl, pos_pad, pos_colsT, feat2, w2flat, n)[:n] + b2
    wei = jax.nn.sigmoid(_bn(xl2, g2, be2))

    res = 2.0 * xs * wei + 2.0 * ys * (1.0 - wei)
    return jnp.zeros((n, x.shape[1]), res.dtype).at[order].set(res)
